# baseline (speedup 1.0000x reference)
import functools

import jax
import jax.numpy as jnp
import numpy as np
from jax import lax
from jax.experimental import pallas as pl
from jax.experimental.pallas import tpu as pltpu
from jax.experimental.pallas import tpu_sc as plsc

_V = 100000
_D = 64
_B = 4096
_L = 50
_NCLS = 5

_NC = 2
_NS = 16
_NW = _NC * _NS
_BPW = _B // _NW
_RPW = _BPW * _L
_CHUNK = 128
_NCH = _RPW // _CHUNK


_NBUF = 5
_NIT = _NCH // _NBUF
_REG = _NS * _BPW


def _sc_body(idx_hbm, tabb_hbm, tabn_hbm, dmap_hbm, zeros_hbm,
             outb_hbm, outn_hbm, idx_v, dmap_v,
             rv0, rv1, rv2, rv3, rv4,
             gs0, gs1, gs2, gs3, gs4, ss0, ss1, ss2, ss3, ss4,
             acc_sh, dsem0, dsem1):
    rows = (rv0, rv1, rv2, rv3, rv4)
    gsem = (gs0, gs1, gs2, gs3, gs4)
    ssem = (ss0, ss1, ss2, ss3, ss4)
    dsem = (dsem0, dsem1)

    s = lax.axis_index("s")
    c = lax.axis_index("c")
    wid = s * _NC + c
    row0 = wid * _RPW
    b0 = wid * _BPW
    a0 = s * _BPW

    pltpu.sync_copy(dmap_hbm.at[s], dmap_v)
    pltpu.sync_copy(idx_hbm.at[pl.ds(row0, _RPW)], idx_v.at[pl.ds(0, _RPW)])
    pltpu.sync_copy(idx_hbm.at[pl.ds(_B * _L + row0, _RPW)],
                    idx_v.at[pl.ds(_RPW, _RPW)])
    pltpu.sync_copy(zeros_hbm, acc_sh.at[pl.ds(a0, _BPW)])
    pltpu.sync_copy(zeros_hbm, acc_sh.at[pl.ds(_REG + a0, _BPW)])

    drains = []
    for t, (tab_hbm, out_hbm) in enumerate(((tabb_hbm, outb_hbm),
                                            (tabn_hbm, outn_hbm))):
        ibase = t * _RPW

        def _gather(ch, slot, tab=tab_hbm, ib=ibase):
            return pltpu.make_async_copy(
                tab.at[idx_v.at[pl.ds(ib + ch * _CHUNK, _CHUNK)]],
                rows[slot], gsem[slot])

        def _scatter(ch, slot, t=t):
            return pltpu.make_async_copy(
                rows[slot], acc_sh.at[dmap_v.at[t * _NCH + ch]], ssem[slot])

        for b in range(_NBUF - 1):
            _gather(b, b).start()

        def _it(it, _, _gather=_gather, _scatter=_scatter):
            for b in range(_NBUF):
                g = it * _NBUF + b
                slot_n = (b - 1) % _NBUF
                if b == 0:
                    @pl.when(it > 0)
                    def _():
                        _scatter(g - 1, slot_n).wait()
                else:
                    _scatter(g - 1, slot_n).wait()
                if b == 0:
                    _gather(g + _NBUF - 1, slot_n).start()
                else:
                    @pl.when(it < _NIT - 1)
                    def _():
                        _gather(g + _NBUF - 1, slot_n).start()
                _gather(g, b).wait()
                _scatter(g, b).start(add=True)
            return 0
        lax.fori_loop(0, _NIT, _it, 0)
        _scatter(_NCH - 1, (_NCH - 1) % _NBUF).wait()

        drain = pltpu.make_async_copy(
            acc_sh.at[pl.ds(t * _REG + a0, _BPW)],
            out_hbm.at[pl.ds(b0, _BPW)], dsem[t])
        drain.start()
        drains.append(drain)

    for drain in drains:
        drain.wait()


_sc_pool = functools.partial(
    pl.kernel,
    mesh=plsc.VectorSubcoreMesh(core_axis_name="c", subcore_axis_name="s"),
    compiler_params=pltpu.CompilerParams(use_tc_tiling_on_sc=False),
    out_type=[
        jax.ShapeDtypeStruct((_B, _D), jnp.float32),
        jax.ShapeDtypeStruct((_B, _D), jnp.float32),
    ],
    scratch_types=(
        [pltpu.VMEM((2 * _RPW,), jnp.int32),
         pltpu.VMEM((2 * _NCH, _CHUNK), jnp.int32)]
        + [pltpu.VMEM((_CHUNK, _D), jnp.float32)] * _NBUF
        + [pltpu.SemaphoreType.DMA] * (2 * _NBUF)
        + [pltpu.VMEM_SHARED((2 * _REG, _D), jnp.float32)]
        + [pltpu.SemaphoreType.DMA] * 2
    ),
)(_sc_body)


_BLK = 512


def _mlp_body(xb_ref, xn_ref, w1a_ref, w1b_ref, b1_ref, w2_ref, b2_ref, o_ref):
    inv_l = jnp.float32(1.0 / _L)
    xb = xb_ref[...] * inv_l
    xn = xn_ref[...] * inv_l
    h = (jnp.dot(xb, w1a_ref[...], preferred_element_type=jnp.float32)
         + jnp.dot(xn, w1b_ref[...], preferred_element_type=jnp.float32)
         + b1_ref[...])
    h = jnp.maximum(h, 0.0)
    logits = (jnp.dot(h, w2_ref[...], preferred_element_type=jnp.float32)
              + b2_ref[...])
    m = jnp.max(logits, axis=1, keepdims=True)
    e = jnp.exp(logits - m)
    p = e / jnp.sum(e, axis=1, keepdims=True)
    o_ref[...] = p[:, :_NCLS]


def _mlp(xb, xn, w1a, w1b, b1, w2, b2):
    grid = (_B // _BLK,)
    return pl.pallas_call(
        _mlp_body,
        grid=grid,
        in_specs=[
            pl.BlockSpec((_BLK, _D), lambda i: (i, 0)),
            pl.BlockSpec((_BLK, _D), lambda i: (i, 0)),
            pl.BlockSpec((_D, 128), lambda i: (0, 0)),
            pl.BlockSpec((_D, 128), lambda i: (0, 0)),
            pl.BlockSpec((1, 128), lambda i: (0, 0)),
            pl.BlockSpec((128, 128), lambda i: (0, 0)),
            pl.BlockSpec((1, 128), lambda i: (0, 0)),
        ],
        out_specs=pl.BlockSpec((_BLK, _NCLS), lambda i: (i, 0)),
        out_shape=jax.ShapeDtypeStruct((_B, _NCLS), jnp.float32),
    )(xb, xn, w1a, w1b, b1, w2, b2)


_seg = np.arange(_RPW, dtype=np.int32) // _L
_seg2 = np.concatenate([_seg, _seg + _REG])
_DMAP = (np.arange(_NS, dtype=np.int32)[:, None] * _BPW
         + _seg2[None, :]).reshape(_NS, 2 * _NCH, _CHUNK)
_ZEROS = np.zeros((_BPW, _D), np.float32)


def kernel(inputs, embed_bow, embed_n_gram, fc1_w, fc1_b, fc2_w, fc2_b):
    idx_all = inputs.reshape(-1).astype(jnp.int32)

    sum_bow, sum_ng = _sc_pool(idx_all, embed_bow, embed_n_gram,
                               _DMAP, _ZEROS)

    w1a = jnp.zeros((_D, 128), jnp.float32).at[:, :_D].set(fc1_w[:, :_D].T)
    w1b = jnp.zeros((_D, 128), jnp.float32).at[:, :_D].set(fc1_w[:, _D:].T)
    b1 = jnp.zeros((1, 128), jnp.float32).at[0, :_D].set(fc1_b)
    w2 = jnp.zeros((128, 128), jnp.float32).at[:_D, :_NCLS].set(fc2_w.T)
    b2 = jnp.full((1, 128), -1e30, jnp.float32).at[0, :_NCLS].set(fc2_b)

    return _mlp(sum_bow, sum_ng, w1a, w1b, b1, w2, b2)


# speedup vs baseline: 1.1569x; 1.1569x over previous
import functools

import jax
import jax.numpy as jnp
import numpy as np
from jax import lax
from jax.experimental import pallas as pl
from jax.experimental.pallas import tpu as pltpu
from jax.experimental.pallas import tpu_sc as plsc

_V = 100000
_D = 64
_B = 4096
_L = 50
_NCLS = 5

_NC = 2
_NS = 16
_NW = _NC * _NS
_BPW = _B // _NW
_RPW = _BPW * _L
_CHUNK = 128
_NCH = _RPW // _CHUNK


_NBUF = 5
_NIT = _NCH // _NBUF
_REG = _NS * _BPW


def _sc_body(toff, idx_hbm, tab_hbm, dmap_hbm, zeros_hbm,
             out_hbm, idx_v, dmap_v,
             rv0, rv1, rv2, rv3, rv4,
             gs0, gs1, gs2, gs3, gs4, ss0, ss1, ss2, ss3, ss4,
             acc_sh, dsem):
    rows = (rv0, rv1, rv2, rv3, rv4)
    gsem = (gs0, gs1, gs2, gs3, gs4)
    ssem = (ss0, ss1, ss2, ss3, ss4)

    s = lax.axis_index("s")
    c = lax.axis_index("c")
    wid = s * _NC + c
    row0 = wid * _RPW
    b0 = wid * _BPW
    a0 = s * _BPW

    pltpu.sync_copy(dmap_hbm.at[s], dmap_v)
    pltpu.sync_copy(idx_hbm.at[pl.ds(toff + row0, _RPW)], idx_v)
    pltpu.sync_copy(zeros_hbm, acc_sh.at[pl.ds(a0, _BPW)])

    def _gather(ch, slot):
        return pltpu.make_async_copy(
            tab_hbm.at[idx_v.at[pl.ds(ch * _CHUNK, _CHUNK)]],
            rows[slot], gsem[slot])

    def _scatter(ch, slot):
        return pltpu.make_async_copy(
            rows[slot], acc_sh.at[dmap_v.at[ch]], ssem[slot])

    for b in range(_NBUF - 1):
        _gather(b, b).start()

    def _it(it, _):
        for b in range(_NBUF):
            g = it * _NBUF + b
            slot_n = (b - 1) % _NBUF
            if b == 0:
                @pl.when(it > 0)
                def _():
                    _scatter(g - 1, slot_n).wait()
            else:
                _scatter(g - 1, slot_n).wait()
            if b == 0:
                _gather(g + _NBUF - 1, slot_n).start()
            else:
                @pl.when(it < _NIT - 1)
                def _():
                    _gather(g + _NBUF - 1, slot_n).start()
            _gather(g, b).wait()
            _scatter(g, b).start(add=True)
        return 0
    lax.fori_loop(0, _NIT, _it, 0)
    _scatter(_NCH - 1, (_NCH - 1) % _NBUF).wait()

    drain = pltpu.make_async_copy(
        acc_sh.at[pl.ds(a0, _BPW)], out_hbm.at[pl.ds(b0, _BPW)], dsem)
    drain.start()
    drain.wait()


def _make_sc(toff):
    return pl.kernel(
        functools.partial(_sc_body, toff),
        mesh=plsc.VectorSubcoreMesh(core_axis_name="c", subcore_axis_name="s"),
        compiler_params=pltpu.CompilerParams(use_tc_tiling_on_sc=False),
        out_type=jax.ShapeDtypeStruct((_B, _D), jnp.float32),
        scratch_types=(
            [pltpu.VMEM((_RPW,), jnp.int32),
             pltpu.VMEM((_NCH, _CHUNK), jnp.int32)]
            + [pltpu.VMEM((_CHUNK, _D), jnp.float32)] * _NBUF
            + [pltpu.SemaphoreType.DMA] * (2 * _NBUF)
            + [pltpu.VMEM_SHARED((_REG, _D), jnp.float32)]
            + [pltpu.SemaphoreType.DMA]
        ),
    )


_sc_bow = _make_sc(0)
_sc_ng = _make_sc(_B * _L)


_BLK = 512


def _mlp_body(xb_ref, xn_ref, w1a_ref, w1b_ref, b1_ref, w2_ref, b2_ref, o_ref):
    inv_l = jnp.float32(1.0 / _L)
    xb = xb_ref[...] * inv_l
    xn = xn_ref[...] * inv_l
    h = (jnp.dot(xb, w1a_ref[...], preferred_element_type=jnp.float32)
         + jnp.dot(xn, w1b_ref[...], preferred_element_type=jnp.float32)
         + b1_ref[...])
    h = jnp.maximum(h, 0.0)
    logits = (jnp.dot(h, w2_ref[...], preferred_element_type=jnp.float32)
              + b2_ref[...])
    m = jnp.max(logits, axis=1, keepdims=True)
    e = jnp.exp(logits - m)
    p = e / jnp.sum(e, axis=1, keepdims=True)
    o_ref[...] = p[:, :_NCLS]


def _mlp(xb, xn, w1a, w1b, b1, w2, b2):
    grid = (_B // _BLK,)
    return pl.pallas_call(
        _mlp_body,
        grid=grid,
        in_specs=[
            pl.BlockSpec((_BLK, _D), lambda i: (i, 0)),
            pl.BlockSpec((_BLK, _D), lambda i: (i, 0)),
            pl.BlockSpec((_D, 128), lambda i: (0, 0)),
            pl.BlockSpec((_D, 128), lambda i: (0, 0)),
            pl.BlockSpec((1, 128), lambda i: (0, 0)),
            pl.BlockSpec((128, 128), lambda i: (0, 0)),
            pl.BlockSpec((1, 128), lambda i: (0, 0)),
        ],
        out_specs=pl.BlockSpec((_BLK, _NCLS), lambda i: (i, 0)),
        out_shape=jax.ShapeDtypeStruct((_B, _NCLS), jnp.float32),
    )(xb, xn, w1a, w1b, b1, w2, b2)


_seg = np.arange(_RPW, dtype=np.int32) // _L
_DMAP = (np.arange(_NS, dtype=np.int32)[:, None] * _BPW
         + _seg[None, :]).reshape(_NS, _NCH, _CHUNK)
_ZEROS = np.zeros((_BPW, _D), np.float32)


def kernel(inputs, embed_bow, embed_n_gram, fc1_w, fc1_b, fc2_w, fc2_b):
    idx_all = inputs.reshape(-1).astype(jnp.int32)

    sum_bow = _sc_bow(idx_all, embed_bow, _DMAP, _ZEROS)
    sum_ng = _sc_ng(idx_all, embed_n_gram, _DMAP, _ZEROS)

    w1a = jnp.zeros((_D, 128), jnp.float32).at[:, :_D].set(fc1_w[:, :_D].T)
    w1b = jnp.zeros((_D, 128), jnp.float32).at[:, :_D].set(fc1_w[:, _D:].T)
    b1 = jnp.zeros((1, 128), jnp.float32).at[0, :_D].set(fc1_b)
    w2 = jnp.zeros((128, 128), jnp.float32).at[:_D, :_NCLS].set(fc2_w.T)
    b2 = jnp.full((1, 128), -1e30, jnp.float32).at[0, :_NCLS].set(fc2_b)

    return _mlp(sum_bow, sum_ng, w1a, w1b, b1, w2, b2)


# speedup vs baseline: 1.1644x; 1.0065x over previous
import functools

import jax
import jax.numpy as jnp
import numpy as np
from jax import lax
from jax.experimental import pallas as pl
from jax.experimental.pallas import tpu as pltpu
from jax.experimental.pallas import tpu_sc as plsc

_V = 100000
_D = 64
_B = 4096
_L = 50
_NCLS = 5

_NC = 2
_NS = 16
_NW = _NC * _NS
_BPW = _B // _NW
_RPW = _BPW * _L
_CHUNK = 128
_NCH = _RPW // _CHUNK


_NBUF = 5
_NIT = _NCH // _NBUF
_REG = _NS * _BPW


def _sc_body(toff, idx_hbm, tab_hbm, dmap_hbm, zeros_hbm,
             out_hbm, idx_v, dmap_v,
             rv0, rv1, rv2, rv3, rv4,
             gs0, gs1, gs2, gs3, gs4, ss0, ss1, ss2, ss3, ss4,
             acc_sh, dsem):
    rows = (rv0, rv1, rv2, rv3, rv4)
    gsem = (gs0, gs1, gs2, gs3, gs4)
    ssem = (ss0, ss1, ss2, ss3, ss4)

    s = lax.axis_index("s")
    c = lax.axis_index("c")
    wid = s * _NC + c
    row0 = wid * _RPW
    b0 = wid * _BPW
    a0 = s * _BPW
    acc_w = acc_sh.at[pl.ds(a0, _BPW)]

    pltpu.sync_copy(dmap_hbm, dmap_v)
    pltpu.sync_copy(idx_hbm.at[pl.ds(toff + row0, _RPW)], idx_v)
    pltpu.sync_copy(zeros_hbm, acc_w)

    def _gather(ch, slot):
        return pltpu.make_async_copy(
            tab_hbm.at[idx_v.at[pl.ds(ch * _CHUNK, _CHUNK)]],
            rows[slot], gsem[slot])

    def _scatter(ch, slot):
        return pltpu.make_async_copy(
            rows[slot], acc_w.at[dmap_v.at[ch]], ssem[slot])

    for b in range(_NBUF - 1):
        _gather(b, b).start()

    def _it(it, _):
        for b in range(_NBUF):
            g = it * _NBUF + b
            slot_n = (b - 1) % _NBUF
            if b == 0:
                @pl.when(it > 0)
                def _():
                    _scatter(g - 1, slot_n).wait()
            else:
                _scatter(g - 1, slot_n).wait()
            if b == 0:
                _gather(g + _NBUF - 1, slot_n).start()
            else:
                @pl.when(it < _NIT - 1)
                def _():
                    _gather(g + _NBUF - 1, slot_n).start()
            _gather(g, b).wait()
            _scatter(g, b).start(add=True)
        return 0
    lax.fori_loop(0, _NIT, _it, 0)
    _scatter(_NCH - 1, (_NCH - 1) % _NBUF).wait()

    drain = pltpu.make_async_copy(
        acc_w, out_hbm.at[pl.ds(b0, _BPW)], dsem)
    drain.start()
    drain.wait()


def _make_sc(toff):
    return pl.kernel(
        functools.partial(_sc_body, toff),
        mesh=plsc.VectorSubcoreMesh(core_axis_name="c", subcore_axis_name="s"),
        compiler_params=pltpu.CompilerParams(use_tc_tiling_on_sc=False),
        out_type=jax.ShapeDtypeStruct((_B, _D), jnp.float32),
        scratch_types=(
            [pltpu.VMEM((_RPW,), jnp.int32),
             pltpu.VMEM((_NCH, _CHUNK), jnp.int32)]
            + [pltpu.VMEM((_CHUNK, _D), jnp.float32)] * _NBUF
            + [pltpu.SemaphoreType.DMA] * (2 * _NBUF)
            + [pltpu.VMEM_SHARED((_REG, _D), jnp.float32)]
            + [pltpu.SemaphoreType.DMA]
        ),
    )


_sc_bow = _make_sc(0)
_sc_ng = _make_sc(_B * _L)


_BLK = 4096


def _mlp_body(xb_ref, xn_ref, w1a_ref, w1b_ref, b1_ref, w2_ref, b2_ref, o_ref):
    inv_l = jnp.float32(1.0 / _L)
    xb = xb_ref[...] * inv_l
    xn = xn_ref[...] * inv_l
    h = (jnp.dot(xb, w1a_ref[...], preferred_element_type=jnp.float32)
         + jnp.dot(xn, w1b_ref[...], preferred_element_type=jnp.float32)
         + b1_ref[...])
    h = jnp.maximum(h, 0.0)
    logits = (jnp.dot(h, w2_ref[...], preferred_element_type=jnp.float32)
              + b2_ref[...])
    m = jnp.max(logits, axis=1, keepdims=True)
    e = jnp.exp(logits - m)
    p = e / jnp.sum(e, axis=1, keepdims=True)
    o_ref[...] = p[:, :_NCLS]


def _mlp(xb, xn, w1a, w1b, b1, w2, b2):
    grid = (_B // _BLK,)
    return pl.pallas_call(
        _mlp_body,
        grid=grid,
        in_specs=[
            pl.BlockSpec((_BLK, _D), lambda i: (i, 0)),
            pl.BlockSpec((_BLK, _D), lambda i: (i, 0)),
            pl.BlockSpec((_D, 128), lambda i: (0, 0)),
            pl.BlockSpec((_D, 128), lambda i: (0, 0)),
            pl.BlockSpec((1, 128), lambda i: (0, 0)),
            pl.BlockSpec((128, 128), lambda i: (0, 0)),
            pl.BlockSpec((1, 128), lambda i: (0, 0)),
        ],
        out_specs=pl.BlockSpec((_BLK, _NCLS), lambda i: (i, 0)),
        out_shape=jax.ShapeDtypeStruct((_B, _NCLS), jnp.float32),
    )(xb, xn, w1a, w1b, b1, w2, b2)


_DMAP = (np.arange(_RPW, dtype=np.int32) // _L).reshape(_NCH, _CHUNK)
_ZEROS = np.zeros((_BPW, _D), np.float32)


def kernel(inputs, embed_bow, embed_n_gram, fc1_w, fc1_b, fc2_w, fc2_b):
    idx_all = inputs.reshape(-1).astype(jnp.int32)

    sum_bow = _sc_bow(idx_all, embed_bow, _DMAP, _ZEROS)
    sum_ng = _sc_ng(idx_all, embed_n_gram, _DMAP, _ZEROS)

    w1a = jnp.zeros((_D, 128), jnp.float32).at[:, :_D].set(fc1_w[:, :_D].T)
    w1b = jnp.zeros((_D, 128), jnp.float32).at[:, :_D].set(fc1_w[:, _D:].T)
    b1 = jnp.zeros((1, 128), jnp.float32).at[0, :_D].set(fc1_b)
    w2 = jnp.zeros((128, 128), jnp.float32).at[:_D, :_NCLS].set(fc2_w.T)
    b2 = jnp.full((1, 128), -1e30, jnp.float32).at[0, :_NCLS].set(fc2_b)

    return _mlp(sum_bow, sum_ng, w1a, w1b, b1, w2, b2)
